# Initial kernel scaffold; baseline (speedup 1.0000x reference)
import jax
import jax.numpy as jnp
from jax.experimental import pallas as pl


def kernel(his, x, edges, v, edge_attr, W_emb, b_emb, We1, be1, We2, be2, Wn1, bn1, Wn2, bn2, Wc1, bc1, Wc2, Wv1, bv1, Wv2, bv2):
    raise NotImplementedError("write your pallas kernel here")


# speedup vs baseline: 2.6723x; 2.6723x over previous
import functools

import jax
import jax.numpy as jnp
from jax import lax
from jax.experimental import pallas as pl
from jax.experimental.pallas import tpu as pltpu
from jax.experimental.pallas import tpu_sc as plsc

F32 = jnp.float32
EPS = 1e-8
NW = 32
NB = 1000
EB = 2000


def _chunk(epw):
    for c in range(128, 0, -8):
        if epw % c == 0:
            return c
    return 8


def _mm(a, w):
    return lax.dot_general(a, w, (((1,), (1,)), ((), ())),
                           preferred_element_type=F32)


def _silu(x):
    return x * jax.nn.sigmoid(x)


def _full(shape):
    return pl.BlockSpec(shape, lambda i: tuple(0 for _ in shape))


def _emb_body(his_ref, w_ref, b_ref, o_ref):
    o_ref[...] = _mm(his_ref[...], w_ref[...]) + b_ref[...]


def _emb(his, W_emb, b_emb):
    n, d = his.shape
    hid = W_emb.shape[0]
    return pl.pallas_call(
        _emb_body,
        grid=(n // NB,),
        in_specs=[pl.BlockSpec((NB, d), lambda i: (i, 0)),
                  _full(W_emb.shape), _full(b_emb.shape)],
        out_specs=pl.BlockSpec((NB, hid), lambda i: (i, 0)),
        out_shape=jax.ShapeDtypeStruct((n, hid), F32),
    )(his, W_emb, b_emb)


def _eproj_body(ea_ref, w_ref, b_ref, o_ref):
    o_ref[...] = _mm(ea_ref[...], w_ref[...]) + b_ref[...]


def _eproj(edge_attr, We1e, be1):
    e, de = edge_attr.shape
    hid = We1e.shape[0]
    return pl.pallas_call(
        _eproj_body,
        grid=(e // EB,),
        in_specs=[pl.BlockSpec((EB, de), lambda i: (i, 0)),
                  _full(We1e.shape), _full(be1.shape)],
        out_specs=pl.BlockSpec((EB, hid), lambda i: (i, 0)),
        out_shape=jax.ShapeDtypeStruct((e, hid), F32),
    )(edge_attr, We1e, be1)


def _proj_body(h_ref, x_ref, wr_ref, wc_ref, tr_ref, tc_ref):
    hid = wr_ref.shape[0]
    dp = x_ref.shape[1]
    hh = h_ref[...]
    xx = x_ref[...]
    zero = jnp.zeros((hh.shape[0], tr_ref.shape[1] - hid - dp), F32)
    tr_ref[:, :hid] = _mm(hh, wr_ref[...])
    tr_ref[:, hid:hid + dp] = xx
    tr_ref[:, hid + dp:] = zero
    tc_ref[:, :hid] = _mm(hh, wc_ref[...])
    tc_ref[:, hid:hid + dp] = xx
    tc_ref[:, hid + dp:] = zero


def _proj(h, xpad, We1r, We1c):
    n, hid = h.shape
    dt = 2 * hid
    out = jax.ShapeDtypeStruct((n, dt), F32)
    return pl.pallas_call(
        _proj_body,
        grid=(n // NB,),
        in_specs=[pl.BlockSpec((NB, hid), lambda i: (i, 0)),
                  pl.BlockSpec((NB, xpad.shape[1]), lambda i: (i, 0)),
                  _full(We1r.shape), _full(We1c.shape)],
        out_specs=(pl.BlockSpec((NB, dt), lambda i: (i, 0)),
                   pl.BlockSpec((NB, dt), lambda i: (i, 0))),
        out_shape=(out, out),
    )(h, xpad, We1r, We1c)


def _edge_body(gtr_ref, gtc_ref, ep_ref, we2_ref, be2_ref, wc1_ref, bc1_ref,
               wc2_ref, wrad_ref, m_ref, tp_ref):
    hid = we2_ref.shape[0]
    tr = gtr_ref[...]
    tc = gtc_ref[...]
    dvec = tr[:, hid:hid + 16] - tc[:, hid:hid + 16]
    radial = _mm(dvec * dvec, jnp.ones((hid, 16), F32))
    pre = tr[:, :hid] + tc[:, :hid] + ep_ref[...] + radial * wrad_ref[...]
    a1 = _silu(pre)
    m = _silu(_mm(a1, we2_ref[...]) + be2_ref[...])
    t = _silu(_mm(m, wc1_ref[...]) + bc1_ref[...])
    s = _mm(t, wc2_ref[...])
    trans = dvec * s
    lane = lax.broadcasted_iota(jnp.int32, trans.shape, 1)
    trans = jnp.where(lane == 3, 1.0, trans)
    m_ref[...] = m
    tp_ref[:, :16] = trans
    tp_ref[:, 16:] = jnp.zeros((trans.shape[0], tp_ref.shape[1] - 16), F32)


def _edge(gTr, gTc, Eproj, We2, be2, Wc1, bc1, Wc2, wrad):
    e, dt = gTr.shape
    hid = We2.shape[0]
    return pl.pallas_call(
        _edge_body,
        grid=(e // EB,),
        in_specs=[pl.BlockSpec((EB, dt), lambda i: (i, 0)),
                  pl.BlockSpec((EB, dt), lambda i: (i, 0)),
                  pl.BlockSpec((EB, hid), lambda i: (i, 0)),
                  _full(We2.shape), _full(be2.shape), _full(Wc1.shape),
                  _full(bc1.shape), _full(Wc2.shape), _full(wrad.shape)],
        out_specs=(pl.BlockSpec((EB, hid), lambda i: (i, 0)),
                   pl.BlockSpec((EB, hid), lambda i: (i, 0))),
        out_shape=(jax.ShapeDtypeStruct((e, hid), F32),
                   jax.ShapeDtypeStruct((e, hid), F32)),
    )(gTr, gTc, Eproj, We2, be2, Wc1, bc1, Wc2, wrad)


def _update_body(h_ref, agg_ref, accp_ref, v_ref, x_ref, wv1_ref, bv1_ref,
                 wv2_ref, bv2_ref, wn1a_ref, wn1b_ref, bn1_ref, wn2_ref,
                 bn2_ref, ho_ref, vo_ref, xo_ref):
    hid = wv1_ref.shape[0]
    dp = v_ref.shape[1]
    hh = h_ref[...]
    agg = agg_ref[0] + agg_ref[1]
    accf = (accp_ref[0] + accp_ref[1])[:, :dp]
    lane = lax.broadcasted_iota(jnp.int32, accf.shape, 1)
    sel3 = (lax.broadcasted_iota(jnp.int32, (dp, dp), 1) == 3).astype(F32)
    deg = _mm(accf, sel3)
    acc = jnp.where(lane < 3, accf, 0.0) / jnp.clip(deg, 1.0, None)
    pv = _mm(_silu(_mm(hh, wv1_ref[...]) + bv1_ref[...]), wv2_ref[...]) \
        + bv2_ref[...]
    vv = v_ref[...]
    vnorm = jnp.sqrt(_mm(vv * vv, jnp.ones((dp, dp), F32)))
    vnew = vv + acc + pv * (vv / (vnorm + EPS))
    xo_ref[...] = x_ref[...] + vnew
    vo_ref[...] = vnew
    hn = _silu(_mm(hh, wn1a_ref[...]) + _mm(agg, wn1b_ref[...]) + bn1_ref[...])
    ho_ref[...] = _mm(hn, wn2_ref[...]) + bn2_ref[...]


def _update(h, aggP, accP, vpad, xpad, Wv1, bv1, Wv2, bv2, Wn1a, Wn1b, bn1,
            Wn2, bn2):
    n, hid = h.shape
    dt = aggP.shape[2]
    dp = vpad.shape[1]
    return pl.pallas_call(
        _update_body,
        grid=(n // NB,),
        in_specs=[pl.BlockSpec((NB, hid), lambda i: (i, 0)),
                  pl.BlockSpec((2, NB, dt), lambda i: (0, i, 0)),
                  pl.BlockSpec((2, NB, dt), lambda i: (0, i, 0)),
                  pl.BlockSpec((NB, dp), lambda i: (i, 0)),
                  pl.BlockSpec((NB, dp), lambda i: (i, 0)),
                  _full(Wv1.shape), _full(bv1.shape), _full(Wv2.shape),
                  _full(bv2.shape), _full(Wn1a.shape), _full(Wn1b.shape),
                  _full(bn1.shape), _full(Wn2.shape), _full(bn2.shape)],
        out_specs=(pl.BlockSpec((NB, hid), lambda i: (i, 0)),
                   pl.BlockSpec((NB, dp), lambda i: (i, 0)),
                   pl.BlockSpec((NB, dp), lambda i: (i, 0))),
        out_shape=(jax.ShapeDtypeStruct((n, hid), F32),
                   jax.ShapeDtypeStruct((n, dp), F32),
                   jax.ShapeDtypeStruct((n, dp), F32)),
    )(h, aggP, accP, vpad, xpad, Wv1, bv1, Wv2, bv2, Wn1a, Wn1b, bn1, Wn2,
      bn2)


def _sc_mesh():
    return plsc.VectorSubcoreMesh(core_axis_name="c", subcore_axis_name="s")


def _sc_gather(Trow, Tcol, rowi, coli):
    n, dt = Trow.shape
    e = rowi.shape[0]
    epw = e // NW
    ch = _chunk(epw)
    nfull = epw // ch
    out = jax.ShapeDtypeStruct((e, dt), F32)

    @functools.partial(
        pl.kernel, mesh=_sc_mesh(),
        out_type=[out, out],
        scratch_types=[pltpu.VMEM((ch,), jnp.int32),
                       pltpu.VMEM((ch,), jnp.int32),
                       pltpu.VMEM((ch, dt), F32),
                       pltpu.VMEM((ch, dt), F32)])
    def k(trow_h, tcol_h, row_h, col_h, gtr_h, gtc_h, ir, ic, br, bc):
        wid = lax.axis_index("s") * 2 + lax.axis_index("c")
        base = wid * epw

        @pl.loop(0, nfull)
        def _(i):
            off = base + i * ch
            pltpu.sync_copy(row_h.at[pl.ds(off, ch)], ir)
            pltpu.sync_copy(col_h.at[pl.ds(off, ch)], ic)
            pltpu.sync_copy(trow_h.at[ir], br)
            pltpu.sync_copy(tcol_h.at[ic], bc)
            pltpu.sync_copy(br, gtr_h.at[pl.ds(off, ch)])
            pltpu.sync_copy(bc, gtc_h.at[pl.ds(off, ch)])

    return k(Trow, Tcol, rowi, coli)


def _sc_scatter(mt, rowi, zrows):
    e, dt = mt.shape
    n = zrows.shape[0]
    epw = e // NW
    ch = _chunk(epw)
    nfull = epw // ch
    rpt = (n // 16) & ~7
    rlast = n - 15 * rpt

    @functools.partial(
        pl.kernel, mesh=_sc_mesh(),
        out_type=jax.ShapeDtypeStruct((2, n, dt), F32),
        scratch_types=[pltpu.VMEM((ch,), jnp.int32),
                       pltpu.VMEM((ch, dt), F32),
                       pltpu.VMEM_SHARED((n, dt), F32)])
    def k(mt_h, row_h, z_h, out_h, idx, buf, accum):
        cid = lax.axis_index("c")
        sid = lax.axis_index("s")
        wid = sid * 2 + cid

        @pl.when(sid < 15)
        def _():
            pltpu.sync_copy(z_h.at[pl.ds(sid * rpt, rpt)],
                            accum.at[pl.ds(sid * rpt, rpt)])

        @pl.when(sid == 15)
        def _():
            pltpu.sync_copy(z_h.at[pl.ds(15 * rpt, rlast)],
                            accum.at[pl.ds(15 * rpt, rlast)])

        plsc.subcore_barrier()
        base = wid * epw

        @pl.loop(0, nfull)
        def _(i):
            off = base + i * ch
            pltpu.sync_copy(row_h.at[pl.ds(off, ch)], idx)
            pltpu.sync_copy(mt_h.at[pl.ds(off, ch)], buf)
            pltpu.sync_copy(buf, accum.at[idx], add=True)

        plsc.subcore_barrier()

        @pl.when(sid < 15)
        def _():
            pltpu.sync_copy(accum.at[pl.ds(sid * rpt, rpt)],
                            out_h.at[cid, pl.ds(sid * rpt, rpt)])

        @pl.when(sid == 15)
        def _():
            pltpu.sync_copy(accum.at[pl.ds(15 * rpt, rlast)],
                            out_h.at[cid, pl.ds(15 * rpt, rlast)])

    return k(mt, rowi, zrows)


def kernel(his, x, edges, v, edge_attr, W_emb, b_emb, We1, be1, We2, be2,
           Wn1, bn1, Wn2, bn2, Wc1, bc1, Wc2, Wv1, bv1, Wv2, bv2):
    n, hid = his.shape[0], W_emb.shape[0]
    row, col = edges[0], edges[1]

    We1r = We1[:, :hid]
    We1c = We1[:, hid:2 * hid]
    wrad = We1[:, 2 * hid].reshape(1, hid)
    We1e = We1[:, 2 * hid + 1:]
    Wn1a = Wn1[:, :hid]
    Wn1b = Wn1[:, hid:]
    be1_ = be1.reshape(1, hid)
    be2_ = be2.reshape(1, hid)
    bn1_ = bn1.reshape(1, hid)
    bn2_ = bn2.reshape(1, hid)
    bc1_ = bc1.reshape(1, hid)
    bv1_ = bv1.reshape(1, hid)
    wc2rep = jnp.tile(Wc2, (16, 1))
    wv2rep = jnp.tile(Wv2, (16, 1))
    bv2rep = jnp.tile(bv2.reshape(1, 1), (1, 16))

    h = _emb(his, W_emb, b_emb.reshape(1, hid))
    Eproj = _eproj(edge_attr, We1e, be1_)
    xpad = jnp.pad(x, ((0, 0), (0, 13)))
    vpad = jnp.pad(v, ((0, 0), (0, 13)))
    zrows = jnp.zeros((n, hid), F32)

    for _ in range(4):
        Trow, Tcol = _proj(h, xpad, We1r, We1c)
        gTr, gTc = _sc_gather(Trow, Tcol, row, col)
        m, tp = _edge(gTr, gTc, Eproj, We2, be2_, Wc1, bc1_, wc2rep, wrad)
        aggP = _sc_scatter(m, row, zrows)
        accP = _sc_scatter(tp, row, zrows)
        h, vpad, xpad = _update(h, aggP, accP, vpad, xpad, Wv1, bv1_, wv2rep,
                                bv2rep, Wn1a, Wn1b, bn1_, Wn2, bn2_)

    return xpad[:, :3], h


# speedup vs baseline: 4.6835x; 1.7526x over previous
import functools

import jax
import jax.numpy as jnp
from jax import lax
from jax.experimental import pallas as pl
from jax.experimental.pallas import tpu as pltpu
from jax.experimental.pallas import tpu_sc as plsc

F32 = jnp.float32
BF16 = jnp.bfloat16
EPS = 1e-8
NW = 32
NB = 1000
EB = 2000


def _chunk(epw):
    for c in range(128, 0, -8):
        if epw % c == 0:
            return c
    return 8


def _mm(a, w):
    return lax.dot_general(a, w, (((1,), (1,)), ((), ())),
                           preferred_element_type=F32)


def _silu(x):
    return x * jax.nn.sigmoid(x)


def _full(shape):
    return pl.BlockSpec(shape, lambda i: tuple(0 for _ in shape))


def _emb_body(his_ref, w_ref, b_ref, o_ref):
    o_ref[...] = _mm(his_ref[...], w_ref[...]) + b_ref[...]


def _emb(his, W_emb, b_emb):
    n, d = his.shape
    hid = W_emb.shape[0]
    return pl.pallas_call(
        _emb_body,
        grid=(n // NB,),
        in_specs=[pl.BlockSpec((NB, d), lambda i: (i, 0)),
                  _full(W_emb.shape), _full(b_emb.shape)],
        out_specs=pl.BlockSpec((NB, hid), lambda i: (i, 0)),
        out_shape=jax.ShapeDtypeStruct((n, hid), F32),
    )(his, W_emb, b_emb)


def _eproj_body(ea_ref, w_ref, b_ref, o_ref):
    o_ref[...] = (_mm(ea_ref[...], w_ref[...]) + b_ref[...]).astype(BF16)


def _eproj(edge_attr, We1e, be1):
    e, de = edge_attr.shape
    hid = We1e.shape[0]
    return pl.pallas_call(
        _eproj_body,
        grid=(e // EB,),
        in_specs=[pl.BlockSpec((EB, de), lambda i: (i, 0)),
                  _full(We1e.shape), _full(be1.shape)],
        out_specs=pl.BlockSpec((EB, hid), lambda i: (i, 0)),
        out_shape=jax.ShapeDtypeStruct((e, hid), BF16),
    )(edge_attr, We1e, be1)


def _pack_hi(x32):
    return lax.bitcast_convert_type(x32.astype(BF16).astype(F32), jnp.int32)


def _proj_body(h_ref, x_ref, wr_ref, wc_ref, tr_ref, tc_ref):
    hid = wr_ref.shape[0]
    dp = x_ref.shape[1]
    hh = h_ref[...]
    aux = _pack_hi(jnp.concatenate(
        [x_ref[...], jnp.zeros((hh.shape[0], hid - dp), F32)], axis=1))
    tr_ref[...] = aux | lax.shift_right_logical(
        _pack_hi(_mm(hh, wr_ref[...])), 16)
    tc_ref[...] = aux | lax.shift_right_logical(
        _pack_hi(_mm(hh, wc_ref[...])), 16)


def _proj(h, xpad, We1r, We1c):
    n, hid = h.shape
    out = jax.ShapeDtypeStruct((n, hid), jnp.int32)
    return pl.pallas_call(
        _proj_body,
        grid=(n // NB,),
        in_specs=[pl.BlockSpec((NB, hid), lambda i: (i, 0)),
                  pl.BlockSpec((NB, xpad.shape[1]), lambda i: (i, 0)),
                  _full(We1r.shape), _full(We1c.shape)],
        out_specs=(pl.BlockSpec((NB, hid), lambda i: (i, 0)),
                   pl.BlockSpec((NB, hid), lambda i: (i, 0))),
        out_shape=(out, out),
    )(h, xpad, We1r, We1c)


def _edge_body(gtr_ref, gtc_ref, ep_ref, we2_ref, be2_ref, wc1_ref, bc1_ref,
               wc2_ref, wrad_ref, m_ref, tp_ref):
    hid = we2_ref.shape[0]
    wr = gtr_ref[...]
    wc = gtc_ref[...]
    pr = lax.bitcast_convert_type(lax.shift_left(wr, 16), F32)
    pc = lax.bitcast_convert_type(lax.shift_left(wc, 16), F32)
    mhi = jnp.int32(-65536)
    ar = lax.bitcast_convert_type(wr & mhi, F32)
    ac = lax.bitcast_convert_type(wc & mhi, F32)
    dvec = (ar - ac)[:, :16]
    radial = _mm(dvec * dvec, jnp.ones((hid, 16), F32))
    pre = pr + pc + ep_ref[...].astype(F32) + radial * wrad_ref[...]
    a1 = _silu(pre)
    m = _silu(_mm(a1, we2_ref[...]) + be2_ref[...])
    t = _silu(_mm(m, wc1_ref[...]) + bc1_ref[...])
    s = _mm(t, wc2_ref[...])
    trans = dvec * s
    lane = lax.broadcasted_iota(jnp.int32, trans.shape, 1)
    trans = jnp.where(lane == 3, 1.0, trans)
    m_ref[...] = m
    tp_ref[:, :16] = trans
    tp_ref[:, 16:] = jnp.zeros((trans.shape[0], tp_ref.shape[1] - 16), F32)


def _edge(gTr, gTc, Eproj, We2, be2, Wc1, bc1, Wc2, wrad):
    e, dt = gTr.shape
    hid = We2.shape[0]
    return pl.pallas_call(
        _edge_body,
        grid=(e // EB,),
        in_specs=[pl.BlockSpec((EB, dt), lambda i: (i, 0)),
                  pl.BlockSpec((EB, dt), lambda i: (i, 0)),
                  pl.BlockSpec((EB, hid), lambda i: (i, 0)),
                  _full(We2.shape), _full(be2.shape), _full(Wc1.shape),
                  _full(bc1.shape), _full(Wc2.shape), _full(wrad.shape)],
        out_specs=(pl.BlockSpec((EB, hid), lambda i: (i, 0)),
                   pl.BlockSpec((EB, hid), lambda i: (i, 0))),
        out_shape=(jax.ShapeDtypeStruct((e, hid), F32),
                   jax.ShapeDtypeStruct((e, hid), F32)),
    )(gTr, gTc, Eproj, We2, be2, Wc1, bc1, Wc2, wrad)


def _update_body(h_ref, agg_ref, accp_ref, v_ref, x_ref, wv1_ref, bv1_ref,
                 wv2_ref, bv2_ref, wn1a_ref, wn1b_ref, bn1_ref, wn2_ref,
                 bn2_ref, ho_ref, vo_ref, xo_ref):
    hid = wv1_ref.shape[0]
    dp = v_ref.shape[1]
    hh = h_ref[...]
    agg = agg_ref[0] + agg_ref[1]
    accf = (accp_ref[0] + accp_ref[1])[:, :dp]
    lane = lax.broadcasted_iota(jnp.int32, accf.shape, 1)
    sel3 = (lax.broadcasted_iota(jnp.int32, (dp, dp), 1) == 3).astype(F32)
    deg = _mm(accf, sel3)
    acc = jnp.where(lane < 3, accf, 0.0) / jnp.clip(deg, 1.0, None)
    pv = _mm(_silu(_mm(hh, wv1_ref[...]) + bv1_ref[...]), wv2_ref[...]) \
        + bv2_ref[...]
    vv = v_ref[...]
    vnorm = jnp.sqrt(_mm(vv * vv, jnp.ones((dp, dp), F32)))
    vnew = vv + acc + pv * (vv / (vnorm + EPS))
    xo_ref[...] = x_ref[...] + vnew
    vo_ref[...] = vnew
    hn = _silu(_mm(hh, wn1a_ref[...]) + _mm(agg, wn1b_ref[...]) + bn1_ref[...])
    ho_ref[...] = _mm(hn, wn2_ref[...]) + bn2_ref[...]


def _update(h, aggP, accP, vpad, xpad, Wv1, bv1, Wv2, bv2, Wn1a, Wn1b, bn1,
            Wn2, bn2):
    n, hid = h.shape
    dt = aggP.shape[2]
    dp = vpad.shape[1]
    return pl.pallas_call(
        _update_body,
        grid=(n // NB,),
        in_specs=[pl.BlockSpec((NB, hid), lambda i: (i, 0)),
                  pl.BlockSpec((2, NB, dt), lambda i: (0, i, 0)),
                  pl.BlockSpec((2, NB, dt), lambda i: (0, i, 0)),
                  pl.BlockSpec((NB, dp), lambda i: (i, 0)),
                  pl.BlockSpec((NB, dp), lambda i: (i, 0)),
                  _full(Wv1.shape), _full(bv1.shape), _full(Wv2.shape),
                  _full(bv2.shape), _full(Wn1a.shape), _full(Wn1b.shape),
                  _full(bn1.shape), _full(Wn2.shape), _full(bn2.shape)],
        out_specs=(pl.BlockSpec((NB, hid), lambda i: (i, 0)),
                   pl.BlockSpec((NB, dp), lambda i: (i, 0)),
                   pl.BlockSpec((NB, dp), lambda i: (i, 0))),
        out_shape=(jax.ShapeDtypeStruct((n, hid), F32),
                   jax.ShapeDtypeStruct((n, dp), F32),
                   jax.ShapeDtypeStruct((n, dp), F32)),
    )(h, aggP, accP, vpad, xpad, Wv1, bv1, Wv2, bv2, Wn1a, Wn1b, bn1, Wn2,
      bn2)


def _sc_mesh():
    return plsc.VectorSubcoreMesh(core_axis_name="c", subcore_axis_name="s")


def _sc_gather(Trow, Tcol, rowi, coli):
    n, dt = Trow.shape
    e = rowi.shape[0]
    epw = e // NW
    ch = _chunk(epw)
    nfull = epw // ch
    npair = nfull // 2
    tail = nfull - 2 * npair
    out = jax.ShapeDtypeStruct((e, dt), jnp.int32)

    @functools.partial(
        pl.kernel, mesh=_sc_mesh(),
        out_type=[out, out],
        scratch_types=[pltpu.VMEM((epw,), jnp.int32),
                       pltpu.VMEM((epw,), jnp.int32),
                       pltpu.VMEM((ch, dt), jnp.int32),
                       pltpu.VMEM((ch, dt), jnp.int32),
                       pltpu.VMEM((ch, dt), jnp.int32),
                       pltpu.VMEM((ch, dt), jnp.int32),
                       pltpu.SemaphoreType.DMA,
                       pltpu.SemaphoreType.DMA,
                       pltpu.SemaphoreType.DMA,
                       pltpu.SemaphoreType.DMA])
    def k(trow_h, tcol_h, row_h, col_h, gtr_h, gtc_h,
          ridx, cidx, ar, ac, br, bc, sga, sgb, swa, swb):
        wid = lax.axis_index("s") * 2 + lax.axis_index("c")
        base = wid * epw
        pltpu.sync_copy(row_h.at[pl.ds(base, epw)], ridx)
        pltpu.sync_copy(col_h.at[pl.ds(base, epw)], cidx)

        @pl.loop(0, npair)
        def _(j):
            i0 = 2 * j
            i1 = i0 + 1
            g1 = pltpu.async_copy(
                trow_h.at[ridx.at[pl.ds(i0 * ch, ch)]], ar, sga)
            g2 = pltpu.async_copy(
                tcol_h.at[cidx.at[pl.ds(i0 * ch, ch)]], ac, sga)
            g3 = pltpu.async_copy(
                trow_h.at[ridx.at[pl.ds(i1 * ch, ch)]], br, sgb)
            g4 = pltpu.async_copy(
                tcol_h.at[cidx.at[pl.ds(i1 * ch, ch)]], bc, sgb)
            g1.wait()
            g2.wait()
            w1 = pltpu.async_copy(ar, gtr_h.at[pl.ds(base + i0 * ch, ch)],
                                  swa)
            w2 = pltpu.async_copy(ac, gtc_h.at[pl.ds(base + i0 * ch, ch)],
                                  swa)
            g3.wait()
            g4.wait()
            w3 = pltpu.async_copy(br, gtr_h.at[pl.ds(base + i1 * ch, ch)],
                                  swb)
            w4 = pltpu.async_copy(bc, gtc_h.at[pl.ds(base + i1 * ch, ch)],
                                  swb)
            w1.wait()
            w2.wait()
            w3.wait()
            w4.wait()

        if tail:
            i0 = 2 * npair
            off = base + i0 * ch
            pltpu.sync_copy(trow_h.at[ridx.at[pl.ds(i0 * ch, ch)]], ar)
            pltpu.sync_copy(tcol_h.at[cidx.at[pl.ds(i0 * ch, ch)]], ac)
            pltpu.sync_copy(ar, gtr_h.at[pl.ds(off, ch)])
            pltpu.sync_copy(ac, gtc_h.at[pl.ds(off, ch)])

    return k(Trow, Tcol, rowi, coli)


def _sc_scatter(mt, rowi, zrows):
    e, dt = mt.shape
    n = zrows.shape[0]
    epw = e // NW
    ch = _chunk(epw)
    nfull = epw // ch
    rpt = (n // 16) & ~7
    rlast = n - 15 * rpt

    npair = nfull // 2
    tail = nfull - 2 * npair

    @functools.partial(
        pl.kernel, mesh=_sc_mesh(),
        out_type=jax.ShapeDtypeStruct((2, n, dt), F32),
        scratch_types=[pltpu.VMEM((ch,), jnp.int32),
                       pltpu.VMEM((ch,), jnp.int32),
                       pltpu.VMEM((ch, dt), F32),
                       pltpu.VMEM((ch, dt), F32),
                       pltpu.VMEM_SHARED((n, dt), F32),
                       pltpu.SemaphoreType.DMA,
                       pltpu.SemaphoreType.DMA])
    def k(mt_h, row_h, z_h, out_h, idxa, idxb, bufa, bufb, accum, sla, slb):
        cid = lax.axis_index("c")
        sid = lax.axis_index("s")
        wid = sid * 2 + cid

        @pl.when(sid < 15)
        def _():
            pltpu.sync_copy(z_h.at[pl.ds(sid * rpt, rpt)],
                            accum.at[pl.ds(sid * rpt, rpt)])

        @pl.when(sid == 15)
        def _():
            pltpu.sync_copy(z_h.at[pl.ds(15 * rpt, rlast)],
                            accum.at[pl.ds(15 * rpt, rlast)])

        plsc.subcore_barrier()
        base = wid * epw

        @pl.loop(0, npair)
        def _(j):
            i0 = 2 * j
            i1 = i0 + 1
            la1 = pltpu.async_copy(row_h.at[pl.ds(base + i0 * ch, ch)],
                                   idxa, sla)
            la2 = pltpu.async_copy(mt_h.at[pl.ds(base + i0 * ch, ch)],
                                   bufa, sla)
            lb1 = pltpu.async_copy(row_h.at[pl.ds(base + i1 * ch, ch)],
                                   idxb, slb)
            lb2 = pltpu.async_copy(mt_h.at[pl.ds(base + i1 * ch, ch)],
                                   bufb, slb)
            la1.wait()
            la2.wait()
            pltpu.sync_copy(bufa, accum.at[idxa], add=True)
            lb1.wait()
            lb2.wait()
            pltpu.sync_copy(bufb, accum.at[idxb], add=True)

        if tail:
            off = base + 2 * npair * ch
            pltpu.sync_copy(row_h.at[pl.ds(off, ch)], idxa)
            pltpu.sync_copy(mt_h.at[pl.ds(off, ch)], bufa)
            pltpu.sync_copy(bufa, accum.at[idxa], add=True)

        plsc.subcore_barrier()

        @pl.when(sid < 15)
        def _():
            pltpu.sync_copy(accum.at[pl.ds(sid * rpt, rpt)],
                            out_h.at[cid, pl.ds(sid * rpt, rpt)])

        @pl.when(sid == 15)
        def _():
            pltpu.sync_copy(accum.at[pl.ds(15 * rpt, rlast)],
                            out_h.at[cid, pl.ds(15 * rpt, rlast)])

    return k(mt, rowi, zrows)


def kernel(his, x, edges, v, edge_attr, W_emb, b_emb, We1, be1, We2, be2,
           Wn1, bn1, Wn2, bn2, Wc1, bc1, Wc2, Wv1, bv1, Wv2, bv2):
    n, hid = his.shape[0], W_emb.shape[0]
    row, col = edges[0], edges[1]

    We1r = We1[:, :hid]
    We1c = We1[:, hid:2 * hid]
    wrad = We1[:, 2 * hid].reshape(1, hid)
    We1e = We1[:, 2 * hid + 1:]
    Wn1a = Wn1[:, :hid]
    Wn1b = Wn1[:, hid:]
    be1_ = be1.reshape(1, hid)
    be2_ = be2.reshape(1, hid)
    bn1_ = bn1.reshape(1, hid)
    bn2_ = bn2.reshape(1, hid)
    bc1_ = bc1.reshape(1, hid)
    bv1_ = bv1.reshape(1, hid)
    wc2rep = jnp.tile(Wc2, (16, 1))
    wv2rep = jnp.tile(Wv2, (16, 1))
    bv2rep = jnp.tile(bv2.reshape(1, 1), (1, 16))

    h = _emb(his, W_emb, b_emb.reshape(1, hid))
    Eproj = _eproj(edge_attr, We1e, be1_)
    xpad = jnp.pad(x, ((0, 0), (0, 13)))
    vpad = jnp.pad(v, ((0, 0), (0, 13)))
    zrows = jnp.zeros((n, hid), F32)

    for _ in range(4):
        Trow, Tcol = _proj(h, xpad, We1r, We1c)
        gTr, gTc = _sc_gather(Trow, Tcol, row, col)
        m, tp = _edge(gTr, gTc, Eproj, We2, be2_, Wc1, bc1_, wc2rep, wrad)
        aggP = _sc_scatter(m, row, zrows)
        accP = _sc_scatter(tp, row, zrows)
        h, vpad, xpad = _update(h, aggP, accP, vpad, xpad, Wv1, bv1_, wv2rep,
                                bv2rep, Wn1a, Wn1b, bn1_, Wn2, bn2_)

    return xpad[:, :3], h


# speedup vs baseline: 4.8788x; 1.0417x over previous
import functools

import jax
import jax.numpy as jnp
from jax import lax
from jax.experimental import pallas as pl
from jax.experimental.pallas import tpu as pltpu
from jax.experimental.pallas import tpu_sc as plsc

F32 = jnp.float32
BF16 = jnp.bfloat16
EPS = 1e-8
NW = 32
NB = 1000
EB = 2000


def _chunk(epw):
    for c in range(128, 0, -8):
        if epw % c == 0:
            return c
    return 8


def _mm(a, w):
    return lax.dot_general(a, w, (((1,), (1,)), ((), ())),
                           preferred_element_type=F32)


def _silu(x):
    return x * jax.nn.sigmoid(x)


def _full(shape):
    return pl.BlockSpec(shape, lambda i: tuple(0 for _ in shape))


def _emb_body(his_ref, w_ref, b_ref, o_ref):
    o_ref[...] = _mm(his_ref[...], w_ref[...]) + b_ref[...]


def _emb(his, W_emb, b_emb):
    n, d = his.shape
    hid = W_emb.shape[0]
    return pl.pallas_call(
        _emb_body,
        grid=(n // NB,),
        in_specs=[pl.BlockSpec((NB, d), lambda i: (i, 0)),
                  _full(W_emb.shape), _full(b_emb.shape)],
        out_specs=pl.BlockSpec((NB, hid), lambda i: (i, 0)),
        out_shape=jax.ShapeDtypeStruct((n, hid), F32),
    )(his, W_emb, b_emb)


def _eproj_body(ea_ref, w_ref, b_ref, o_ref):
    o_ref[...] = (_mm(ea_ref[...], w_ref[...]) + b_ref[...]).astype(BF16)


def _eproj(edge_attr, We1e, be1):
    e, de = edge_attr.shape
    hid = We1e.shape[0]
    return pl.pallas_call(
        _eproj_body,
        grid=(e // EB,),
        in_specs=[pl.BlockSpec((EB, de), lambda i: (i, 0)),
                  _full(We1e.shape), _full(be1.shape)],
        out_specs=pl.BlockSpec((EB, hid), lambda i: (i, 0)),
        out_shape=jax.ShapeDtypeStruct((e, hid), BF16),
    )(edge_attr, We1e, be1)


def _pack_hi(x32):
    return lax.bitcast_convert_type(x32.astype(BF16).astype(F32), jnp.int32)


def _proj_body(h_ref, x_ref, wr_ref, wc_ref, tr_ref, tc_ref):
    hid = wr_ref.shape[0]
    dp = x_ref.shape[1]
    hh = h_ref[...]
    aux = _pack_hi(jnp.concatenate(
        [x_ref[...], jnp.zeros((hh.shape[0], hid - dp), F32)], axis=1))
    tr_ref[...] = aux | lax.shift_right_logical(
        _pack_hi(_mm(hh, wr_ref[...])), 16)
    tc_ref[...] = aux | lax.shift_right_logical(
        _pack_hi(_mm(hh, wc_ref[...])), 16)


def _proj(h, xpad, We1r, We1c):
    n, hid = h.shape
    out = jax.ShapeDtypeStruct((n, hid), jnp.int32)
    return pl.pallas_call(
        _proj_body,
        grid=(n // NB,),
        in_specs=[pl.BlockSpec((NB, hid), lambda i: (i, 0)),
                  pl.BlockSpec((NB, xpad.shape[1]), lambda i: (i, 0)),
                  _full(We1r.shape), _full(We1c.shape)],
        out_specs=(pl.BlockSpec((NB, hid), lambda i: (i, 0)),
                   pl.BlockSpec((NB, hid), lambda i: (i, 0))),
        out_shape=(out, out),
    )(h, xpad, We1r, We1c)


def _edge_body(gtr_ref, gtc_ref, ep_ref, we2_ref, be2_ref, wc1_ref, bc1_ref,
               wc2_ref, wrad_ref, m_ref, tp_ref):
    hid = we2_ref.shape[0]
    wr = gtr_ref[...]
    wc = gtc_ref[...]
    pr = lax.bitcast_convert_type(lax.shift_left(wr, 16), F32)
    pc = lax.bitcast_convert_type(lax.shift_left(wc, 16), F32)
    mhi = jnp.int32(-65536)
    ar = lax.bitcast_convert_type(wr & mhi, F32)
    ac = lax.bitcast_convert_type(wc & mhi, F32)
    dvec = (ar - ac)[:, :16]
    radial = _mm(dvec * dvec, jnp.ones((hid, 16), F32))
    pre = pr + pc + ep_ref[...].astype(F32) + radial * wrad_ref[...]
    a1 = _silu(pre).astype(BF16)
    m = _silu(_mm(a1, we2_ref[...]) + be2_ref[...])
    t = _silu(_mm(m.astype(BF16), wc1_ref[...]) + bc1_ref[...]).astype(BF16)
    s = _mm(t, wc2_ref[...])
    trans = dvec * s
    lane = lax.broadcasted_iota(jnp.int32, trans.shape, 1)
    trans = jnp.where(lane == 3, 1.0, trans)
    m_ref[...] = m
    tp_ref[:, :16] = trans
    tp_ref[:, 16:] = jnp.zeros((trans.shape[0], tp_ref.shape[1] - 16), F32)


def _edge(gTr, gTc, Eproj, We2, be2, Wc1, bc1, Wc2, wrad):
    e, dt = gTr.shape
    hid = We2.shape[0]
    return pl.pallas_call(
        _edge_body,
        grid=(e // EB,),
        in_specs=[pl.BlockSpec((EB, dt), lambda i: (i, 0)),
                  pl.BlockSpec((EB, dt), lambda i: (i, 0)),
                  pl.BlockSpec((EB, hid), lambda i: (i, 0)),
                  _full(We2.shape), _full(be2.shape), _full(Wc1.shape),
                  _full(bc1.shape), _full(Wc2.shape), _full(wrad.shape)],
        out_specs=(pl.BlockSpec((EB, hid), lambda i: (i, 0)),
                   pl.BlockSpec((EB, hid), lambda i: (i, 0))),
        out_shape=(jax.ShapeDtypeStruct((e, hid), F32),
                   jax.ShapeDtypeStruct((e, hid), F32)),
    )(gTr, gTc, Eproj, We2, be2, Wc1, bc1, Wc2, wrad)


def _update_body(h_ref, agg_ref, accp_ref, v_ref, x_ref, wv1_ref, bv1_ref,
                 wv2_ref, bv2_ref, wn1a_ref, wn1b_ref, bn1_ref, wn2_ref,
                 bn2_ref, ho_ref, vo_ref, xo_ref):
    hid = wv1_ref.shape[0]
    dp = v_ref.shape[1]
    hh = h_ref[...]
    agg = jnp.sum(agg_ref[...], axis=0)
    accf = jnp.sum(accp_ref[...], axis=0)[:, :dp]
    lane = lax.broadcasted_iota(jnp.int32, accf.shape, 1)
    sel3 = (lax.broadcasted_iota(jnp.int32, (dp, dp), 1) == 3).astype(F32)
    deg = _mm(accf, sel3)
    acc = jnp.where(lane < 3, accf, 0.0) / jnp.clip(deg, 1.0, None)
    pv = _mm(_silu(_mm(hh, wv1_ref[...]) + bv1_ref[...]), wv2_ref[...]) \
        + bv2_ref[...]
    vv = v_ref[...]
    vnorm = jnp.sqrt(_mm(vv * vv, jnp.ones((dp, dp), F32)))
    vnew = vv + acc + pv * (vv / (vnorm + EPS))
    xo_ref[...] = x_ref[...] + vnew
    vo_ref[...] = vnew
    hn = _silu(_mm(hh, wn1a_ref[...]) + _mm(agg, wn1b_ref[...]) + bn1_ref[...])
    ho_ref[...] = _mm(hn, wn2_ref[...]) + bn2_ref[...]


def _update(h, aggP, accP, vpad, xpad, Wv1, bv1, Wv2, bv2, Wn1a, Wn1b, bn1,
            Wn2, bn2):
    n, hid = h.shape
    dt = aggP.shape[2]
    dp = vpad.shape[1]
    nparts = aggP.shape[0]
    return pl.pallas_call(
        _update_body,
        grid=(n // NB,),
        in_specs=[pl.BlockSpec((NB, hid), lambda i: (i, 0)),
                  pl.BlockSpec((nparts, NB, dt), lambda i: (0, i, 0)),
                  pl.BlockSpec((nparts, NB, dt), lambda i: (0, i, 0)),
                  pl.BlockSpec((NB, dp), lambda i: (i, 0)),
                  pl.BlockSpec((NB, dp), lambda i: (i, 0)),
                  _full(Wv1.shape), _full(bv1.shape), _full(Wv2.shape),
                  _full(bv2.shape), _full(Wn1a.shape), _full(Wn1b.shape),
                  _full(bn1.shape), _full(Wn2.shape), _full(bn2.shape)],
        out_specs=(pl.BlockSpec((NB, hid), lambda i: (i, 0)),
                   pl.BlockSpec((NB, dp), lambda i: (i, 0)),
                   pl.BlockSpec((NB, dp), lambda i: (i, 0))),
        out_shape=(jax.ShapeDtypeStruct((n, hid), F32),
                   jax.ShapeDtypeStruct((n, dp), F32),
                   jax.ShapeDtypeStruct((n, dp), F32)),
    )(h, aggP, accP, vpad, xpad, Wv1, bv1, Wv2, bv2, Wn1a, Wn1b, bn1, Wn2,
      bn2)


def _sc_mesh():
    return plsc.VectorSubcoreMesh(core_axis_name="c", subcore_axis_name="s")


def _sc_gather(Trow, Tcol, rowi, coli):
    n, dt = Trow.shape
    e = rowi.shape[0]
    epw = e // NW
    ch = _chunk(epw)
    nfull = epw // ch
    npair = nfull // 2
    tail = nfull - 2 * npair
    out = jax.ShapeDtypeStruct((e, dt), jnp.int32)

    @functools.partial(
        pl.kernel, mesh=_sc_mesh(),
        out_type=[out, out],
        scratch_types=[pltpu.VMEM((epw,), jnp.int32),
                       pltpu.VMEM((epw,), jnp.int32),
                       pltpu.VMEM((ch, dt), jnp.int32),
                       pltpu.VMEM((ch, dt), jnp.int32),
                       pltpu.VMEM((ch, dt), jnp.int32),
                       pltpu.VMEM((ch, dt), jnp.int32),
                       pltpu.SemaphoreType.DMA,
                       pltpu.SemaphoreType.DMA,
                       pltpu.SemaphoreType.DMA,
                       pltpu.SemaphoreType.DMA])
    def k(trow_h, tcol_h, row_h, col_h, gtr_h, gtc_h,
          ridx, cidx, ar, ac, br, bc, sga, sgb, swa, swb):
        wid = lax.axis_index("s") * 2 + lax.axis_index("c")
        base = wid * epw
        pltpu.sync_copy(row_h.at[pl.ds(base, epw)], ridx)
        pltpu.sync_copy(col_h.at[pl.ds(base, epw)], cidx)

        @pl.loop(0, npair)
        def _(j):
            i0 = 2 * j
            i1 = i0 + 1
            g1 = pltpu.async_copy(
                trow_h.at[ridx.at[pl.ds(i0 * ch, ch)]], ar, sga)
            g2 = pltpu.async_copy(
                tcol_h.at[cidx.at[pl.ds(i0 * ch, ch)]], ac, sga)
            g3 = pltpu.async_copy(
                trow_h.at[ridx.at[pl.ds(i1 * ch, ch)]], br, sgb)
            g4 = pltpu.async_copy(
                tcol_h.at[cidx.at[pl.ds(i1 * ch, ch)]], bc, sgb)
            g1.wait()
            g2.wait()
            w1 = pltpu.async_copy(ar, gtr_h.at[pl.ds(base + i0 * ch, ch)],
                                  swa)
            w2 = pltpu.async_copy(ac, gtc_h.at[pl.ds(base + i0 * ch, ch)],
                                  swa)
            g3.wait()
            g4.wait()
            w3 = pltpu.async_copy(br, gtr_h.at[pl.ds(base + i1 * ch, ch)],
                                  swb)
            w4 = pltpu.async_copy(bc, gtc_h.at[pl.ds(base + i1 * ch, ch)],
                                  swb)
            w1.wait()
            w2.wait()
            w3.wait()
            w4.wait()

        if tail:
            i0 = 2 * npair
            off = base + i0 * ch
            pltpu.sync_copy(trow_h.at[ridx.at[pl.ds(i0 * ch, ch)]], ar)
            pltpu.sync_copy(tcol_h.at[cidx.at[pl.ds(i0 * ch, ch)]], ac)
            pltpu.sync_copy(ar, gtr_h.at[pl.ds(off, ch)])
            pltpu.sync_copy(ac, gtc_h.at[pl.ds(off, ch)])

    return k(Trow, Tcol, rowi, coli)


def _sc_scatter(mt, rowi, zrows):
    e, dt = mt.shape
    n = zrows.shape[0]
    epw = e // NW
    ch = _chunk(epw)
    nfull = epw // ch
    rpt = (n // 16) & ~7
    rlast = n - 15 * rpt

    npair = nfull // 2
    tail = nfull - 2 * npair

    @functools.partial(
        pl.kernel, mesh=_sc_mesh(),
        out_type=jax.ShapeDtypeStruct((2, n, dt), F32),
        scratch_types=[pltpu.VMEM((ch,), jnp.int32),
                       pltpu.VMEM((ch,), jnp.int32),
                       pltpu.VMEM((ch, dt), F32),
                       pltpu.VMEM((ch, dt), F32),
                       pltpu.VMEM_SHARED((n, dt), F32),
                       pltpu.SemaphoreType.DMA,
                       pltpu.SemaphoreType.DMA])
    def k(mt_h, row_h, z_h, out_h, idxa, idxb, bufa, bufb, accum, sla, slb):
        cid = lax.axis_index("c")
        sid = lax.axis_index("s")
        wid = sid * 2 + cid

        @pl.when(sid < 15)
        def _():
            pltpu.sync_copy(z_h.at[pl.ds(sid * rpt, rpt)],
                            accum.at[pl.ds(sid * rpt, rpt)])

        @pl.when(sid == 15)
        def _():
            pltpu.sync_copy(z_h.at[pl.ds(15 * rpt, rlast)],
                            accum.at[pl.ds(15 * rpt, rlast)])

        plsc.subcore_barrier()
        base = wid * epw

        @pl.loop(0, npair)
        def _(j):
            i0 = 2 * j
            i1 = i0 + 1
            la1 = pltpu.async_copy(row_h.at[pl.ds(base + i0 * ch, ch)],
                                   idxa, sla)
            la2 = pltpu.async_copy(mt_h.at[pl.ds(base + i0 * ch, ch)],
                                   bufa, sla)
            lb1 = pltpu.async_copy(row_h.at[pl.ds(base + i1 * ch, ch)],
                                   idxb, slb)
            lb2 = pltpu.async_copy(mt_h.at[pl.ds(base + i1 * ch, ch)],
                                   bufb, slb)
            la1.wait()
            la2.wait()
            pltpu.sync_copy(bufa, accum.at[idxa], add=True)
            lb1.wait()
            lb2.wait()
            pltpu.sync_copy(bufb, accum.at[idxb], add=True)

        if tail:
            off = base + 2 * npair * ch
            pltpu.sync_copy(row_h.at[pl.ds(off, ch)], idxa)
            pltpu.sync_copy(mt_h.at[pl.ds(off, ch)], bufa)
            pltpu.sync_copy(bufa, accum.at[idxa], add=True)

        plsc.subcore_barrier()

        @pl.when(sid < 15)
        def _():
            pltpu.sync_copy(accum.at[pl.ds(sid * rpt, rpt)],
                            out_h.at[cid, pl.ds(sid * rpt, rpt)])

        @pl.when(sid == 15)
        def _():
            pltpu.sync_copy(accum.at[pl.ds(15 * rpt, rlast)],
                            out_h.at[cid, pl.ds(15 * rpt, rlast)])

    return k(mt, rowi, zrows)


def kernel(his, x, edges, v, edge_attr, W_emb, b_emb, We1, be1, We2, be2,
           Wn1, bn1, Wn2, bn2, Wc1, bc1, Wc2, Wv1, bv1, Wv2, bv2):
    n, hid = his.shape[0], W_emb.shape[0]
    row, col = edges[0], edges[1]

    We1r = We1[:, :hid]
    We1c = We1[:, hid:2 * hid]
    wrad = We1[:, 2 * hid].reshape(1, hid)
    We1e = We1[:, 2 * hid + 1:]
    Wn1a = Wn1[:, :hid]
    Wn1b = Wn1[:, hid:]
    be1_ = be1.reshape(1, hid)
    be2_ = be2.reshape(1, hid)
    bn1_ = bn1.reshape(1, hid)
    bn2_ = bn2.reshape(1, hid)
    bc1_ = bc1.reshape(1, hid)
    bv1_ = bv1.reshape(1, hid)
    wc2rep = jnp.tile(Wc2, (16, 1))
    wv2rep = jnp.tile(Wv2, (16, 1))
    bv2rep = jnp.tile(bv2.reshape(1, 1), (1, 16))

    h = _emb(his, W_emb, b_emb.reshape(1, hid))
    Eproj = _eproj(edge_attr, We1e, be1_)
    xpad = jnp.pad(x, ((0, 0), (0, 13)))
    vpad = jnp.pad(v, ((0, 0), (0, 13)))
    zrows = jnp.zeros((n, hid), F32)
    we2b = We2.astype(BF16)
    wc1b = Wc1.astype(BF16)
    wc2repb = wc2rep.astype(BF16)

    e = row.shape[0]
    esplit = (e * 3 // 5) // (8 * NW) * (8 * NW)
    spans = ((0, esplit), (esplit, e))

    for _ in range(4):
        Trow, Tcol = _proj(h, xpad, We1r, We1c)
        aggs, accs = [], []
        for lo, hi in spans:
            rs, cs = row[lo:hi], col[lo:hi]
            gTr, gTc = _sc_gather(Trow, Tcol, rs, cs)
            m, tp = _edge(gTr, gTc, Eproj[lo:hi], we2b, be2_, wc1b, bc1_,
                          wc2repb, wrad)
            aggs.append(_sc_scatter(m, rs, zrows))
            accs.append(_sc_scatter(tp, rs, zrows))
        aggP = jnp.concatenate(aggs, axis=0)
        accP = jnp.concatenate(accs, axis=0)
        h, vpad, xpad = _update(h, aggP, accP, vpad, xpad, Wv1, bv1_, wv2rep,
                                bv2rep, Wn1a, Wn1b, bn1_, Wn2, bn2_)

    return xpad[:, :3], h
